# baseline (speedup 1.0000x reference)
import functools

import jax
import jax.numpy as jnp
from jax import lax
from jax.experimental import pallas as pl
from jax.experimental.pallas import tpu as pltpu
from jax.experimental.pallas import tpu_sc as plsc

_NC = 2
_NS = 16
_NW = _NC * _NS

_DP = 104
_JT = _DP // 8
_NB = 4
_RS = 112
_REP = 561


@functools.lru_cache(maxsize=None)
def _build(B, V, D):
    n_ib = B // 128
    ib_per_w = n_ib // _NW
    n_steps = ib_per_w // _NB

    mesh = plsc.VectorSubcoreMesh(core_axis_name="c", subcore_axis_name="s")

    @functools.partial(
        pl.kernel,
        out_type=jax.ShapeDtypeStruct((_JT, n_ib, 8, 128), jnp.float32),
        mesh=mesh,
        compiler_params=pltpu.CompilerParams(
            use_tc_tiling_on_sc=False, needs_layout_passes=False),
        scratch_types=[
            pltpu.VMEM((2, _NB, 128), jnp.int32),
            pltpu.VMEM((8, 128), jnp.float32),
            pltpu.VMEM((16 * _REP,), jnp.float32),
            pltpu.VMEM((_NB, 8, 128), jnp.float32),
            pltpu.VMEM((_NB, 8, 128), jnp.float32),
            pltpu.VMEM((_NB, 8, 128), jnp.float32),
            pltpu.VMEM((_NB, 8, 128), jnp.float32),
            pltpu.SemaphoreType.DMA,
            pltpu.SemaphoreType.DMA,
            pltpu.SemaphoreType.DMA,
            pltpu.SemaphoreType.DMA,
            pltpu.SemaphoreType.DMA,
        ],
    )
    def emb(idx_hbm, table_hbm, out_hbm, idxb, tblv, rep,
            tb0, tb1, tb2, tb3, so0, so1, so2, so3, semi):
        tbufs = (tb0, tb1, tb2, tb3)
        sems = (so0, so1, so2, so3)
        w = lax.axis_index("s") * _NC + lax.axis_index("c")
        col_base = w * ib_per_w
        lane_off = lax.iota(jnp.int32, 16) * _REP

        pltpu.sync_copy(table_hbm, tblv)
        for v in range(V):
            for k in range(7):
                val = tblv[v, pl.ds(16 * k, 16)]
                for r in range(16):
                    rep[pl.ds(r * _REP + v * _RS + 16 * k, 16)] = val

        pltpu.async_copy(idx_hbm.at[pl.ds(col_base, _NB)], idxb.at[0], semi)

        def step_body(step, _):
            par = lax.rem(step, 2)

            pltpu.make_async_copy(
                idx_hbm.at[pl.ds(0, _NB)], idxb.at[par], semi).wait()

            @pl.when(step + 1 < n_steps)
            def _():
                pltpu.async_copy(
                    idx_hbm.at[pl.ds(col_base + (step + 1) * _NB, _NB)],
                    idxb.at[1 - par], semi)

            addrs = []
            for iv in range(8 * _NB):
                idxv = idxb[par, iv // 8, pl.ds(16 * (iv % 8), 16)]
                addrs.append(idxv * _RS + lane_off)

            def jt_body(jt, _):
                gjt = step * _JT + jt
                m = lax.rem(jt, 4)
                for mi in range(4):
                    @pl.when(m == mi)
                    def _(tb=tbufs[mi], sem=sems[mi]):
                        @pl.when(gjt >= 4)
                        def _():
                            pltpu.make_async_copy(
                                tb, out_hbm.at[0, pl.ds(0, _NB)], sem).wait()
                        for s in range(8):
                            joff = jt * 8 + s
                            prev = None
                            for g in range(_NB):
                                vals = [plsc.load_gather(
                                            rep, [addrs[8 * g + t] + joff])
                                        for t in range(8)]
                                if prev is not None:
                                    gp, pv = prev
                                    for t in range(8):
                                        iv = 8 * gp + t
                                        tb[iv // 8, s,
                                           pl.ds(16 * (iv % 8), 16)] = pv[t]
                                prev = (g, vals)
                            gp, pv = prev
                            for t in range(8):
                                iv = 8 * gp + t
                                tb[iv // 8, s,
                                   pl.ds(16 * (iv % 8), 16)] = pv[t]
                        pltpu.async_copy(
                            tb,
                            out_hbm.at[jt, pl.ds(col_base + step * _NB, _NB)],
                            sem)
                return 0

            lax.fori_loop(0, _JT, jt_body, 0)
            return 0

        lax.fori_loop(0, n_steps, step_body, 0)

        for mi in range(4):
            pltpu.make_async_copy(
                tbufs[mi], out_hbm.at[0, pl.ds(0, _NB)], sems[mi]).wait()

    return emb


def kernel(indices, table):
    B = indices.shape[0]
    V, D = table.shape
    idx2d = indices.astype(jnp.int32).reshape(B // 128, 128)
    table_pad = jnp.zeros((8, 128), jnp.float32).at[:V, :D].set(table)
    res4 = _build(B, V, D)(idx2d, table_pad)
    out = res4.transpose(1, 3, 0, 2).reshape(B, _DP)[:, :D]
    return out


# speedup vs baseline: 2.2288x; 2.2288x over previous
import functools

import jax
import jax.numpy as jnp
from jax import lax
from jax.experimental import pallas as pl
from jax.experimental.pallas import tpu as pltpu
from jax.experimental.pallas import tpu_sc as plsc

_NC = 2
_NS = 16
_NW = _NC * _NS

_DP = 104
_JT = _DP // 8
_NB = 4
_RS = 112
_REP = 561


@functools.lru_cache(maxsize=None)
def _build(B, V, D):
    n_ib = B // 128
    ib_per_w = n_ib // _NW
    n_steps = ib_per_w // _NB

    mesh = plsc.VectorSubcoreMesh(core_axis_name="c", subcore_axis_name="s")

    @functools.partial(
        pl.kernel,
        out_type=jax.ShapeDtypeStruct((_JT, n_ib, 8, 128), jnp.float32),
        mesh=mesh,
        compiler_params=pltpu.CompilerParams(
            use_tc_tiling_on_sc=False, needs_layout_passes=False),
        scratch_types=[
            pltpu.VMEM((2, _NB, 128), jnp.int32),
            pltpu.VMEM((8, 128), jnp.float32),
            pltpu.VMEM((16 * _REP,), jnp.float32),
            pltpu.VMEM((2, _JT, _NB, 8, 128), jnp.float32),
            pltpu.SemaphoreType.DMA,
            pltpu.SemaphoreType.DMA,
        ],
    )
    def emb(idx_hbm, table_hbm, out_hbm, idxb, tblv, rep, buf, semo, semi):
        w = lax.axis_index("s") * _NC + lax.axis_index("c")
        col_base = w * ib_per_w
        lane_off = lax.iota(jnp.int32, 16) * _REP

        pltpu.sync_copy(table_hbm, tblv)
        for v in range(V):
            for k in range(7):
                val = tblv[v, pl.ds(16 * k, 16)]
                for r in range(16):
                    rep[pl.ds(r * _REP + v * _RS + 16 * k, 16)] = val

        pltpu.async_copy(idx_hbm.at[pl.ds(col_base, _NB)], idxb.at[0], semi)

        def step_body(step, _):
            par = lax.rem(step, 2)

            pltpu.make_async_copy(
                idx_hbm.at[pl.ds(0, _NB)], idxb.at[par], semi).wait()

            @pl.when(step + 1 < n_steps)
            def _():
                pltpu.async_copy(
                    idx_hbm.at[pl.ds(col_base + (step + 1) * _NB, _NB)],
                    idxb.at[1 - par], semi)

            @pl.when(step >= 2)
            def _():
                for jt in range(_JT):
                    pltpu.make_async_copy(
                        buf.at[par, jt],
                        out_hbm.at[jt, pl.ds(0, _NB)], semo).wait()

            addrs = []
            for iv in range(8 * _NB):
                idxv = idxb[par, iv // 8, pl.ds(16 * (iv % 8), 16)]
                addrs.append(idxv * _RS + lane_off)

            def jt_body(jt, _):
                slab = buf.at[par, jt]
                for s in range(8):
                    joff = jt * 8 + s
                    prev = None
                    for g in range(4):
                        vals = [plsc.load_gather(rep,
                                                 [addrs[8 * g + t] + joff])
                                for t in range(8)]
                        if prev is not None:
                            gp, pv = prev
                            for t in range(8):
                                iv = 8 * gp + t
                                slab[iv // 8, s,
                                     pl.ds(16 * (iv % 8), 16)] = pv[t]
                        prev = (g, vals)
                    gp, pv = prev
                    for t in range(8):
                        iv = 8 * gp + t
                        slab[iv // 8, s, pl.ds(16 * (iv % 8), 16)] = pv[t]
                pltpu.async_copy(
                    slab, out_hbm.at[jt, pl.ds(col_base + step * _NB, _NB)],
                    semo)
                return 0

            lax.fori_loop(0, _JT, jt_body, 0)
            return 0

        lax.fori_loop(0, n_steps, step_body, 0)

        for _ in range(2):
            for jt in range(_JT):
                pltpu.make_async_copy(
                    buf.at[0, jt], out_hbm.at[jt, pl.ds(0, _NB)], semo).wait()

    return emb


def kernel(indices, table):
    B = indices.shape[0]
    V, D = table.shape
    idx2d = indices.astype(jnp.int32).reshape(B // 128, 128)
    table_pad = jnp.zeros((8, 128), jnp.float32).at[:V, :D].set(table)
    res4 = _build(B, V, D)(idx2d, table_pad)
    out = res4.transpose(1, 3, 0, 2).reshape(B, _DP)[:, :D]
    return out
